# Initial kernel scaffold; baseline (speedup 1.0000x reference)
import jax
import jax.numpy as jnp
from jax.experimental import pallas as pl


def kernel(x, Emb, W1, b1, W2, b2, Wf, bf):
    raise NotImplementedError("write your pallas kernel here")


# speedup vs baseline: 8.1618x; 8.1618x over previous
import functools

import jax
import jax.numpy as jnp
from jax import lax
from jax.experimental import pallas as pl
from jax.experimental.pallas import tpu as pltpu
from jax.experimental.pallas import tpu_sc as plsc

B, T, V, D = 16384, 100, 1000000, 32
L = 16
NC, NS = 2, 16
NW = NC * NS
ROWS_PER_W = B // NW
CHUNK = 128
ACC_W = 5 * L


def _sc_accumulate(x, emb, wf):
    mesh = plsc.VectorSubcoreMesh(core_axis_name="c", subcore_axis_name="s")

    @functools.partial(
        pl.kernel,
        mesh=mesh,
        compiler_params=pltpu.CompilerParams(use_tc_tiling_on_sc=False),
        out_type=jax.ShapeDtypeStruct((B, ACC_W), jnp.float32),
        scratch_types=[
            pltpu.VMEM((CHUNK, T), jnp.int32),
            pltpu.VMEM((T, D), jnp.float32),
            pltpu.VMEM((T, D), jnp.float32),
            pltpu.VMEM((T, D), jnp.float32),
            pltpu.VMEM((CHUNK, ACC_W), jnp.float32),
            pltpu.SemaphoreType.DMA,
            pltpu.SemaphoreType.DMA,
        ],
    )
    def k(x_hbm, emb_hbm, wf_hbm, out_hbm, idx_v, wf_v, buf0, buf1, out_v,
          sem0, sem1):
        wid = lax.axis_index("s") * NC + lax.axis_index("c")
        base = wid * ROWS_PER_W
        pltpu.sync_copy(wf_hbm, wf_v)

        def compute_row(buf, row):
            def t_body(t, carry):
                s0, s1, q0, q1, r0, r1 = carry
                e0 = buf[t, pl.ds(0, L)]
                e1 = buf[t, pl.ds(L, L)]
                w0 = wf_v[t, pl.ds(0, L)]
                w1 = wf_v[t, pl.ds(L, L)]
                return (s0 + e0, s1 + e1,
                        q0 + e0 * e0, q1 + e1 * e1,
                        r0 + e0 * w0, r1 + e1 * w1)

            z = jnp.zeros((L,), jnp.float32)
            s0, s1, q0, q1, r0, r1 = lax.fori_loop(
                0, T, t_body, (z, z, z, z, z, z))
            out_v[row, pl.ds(0, L)] = s0
            out_v[row, pl.ds(L, L)] = s1
            out_v[row, pl.ds(2 * L, L)] = q0
            out_v[row, pl.ds(3 * L, L)] = q1
            out_v[row, pl.ds(4 * L, L)] = r0 + r1

        def chunk_body(c, _):
            rbase = base + c * CHUNK
            pltpu.sync_copy(x_hbm.at[pl.ds(rbase, CHUNK)], idx_v)
            pltpu.async_copy(emb_hbm.at[idx_v.at[0]], buf0, sem0)

            def pair_body(g, _):
                ra = 2 * g
                rb = ra + 1
                pltpu.async_copy(emb_hbm.at[idx_v.at[rb]], buf1, sem1)
                pltpu.make_async_copy(
                    emb_hbm.at[idx_v.at[ra]], buf0, sem0).wait()
                compute_row(buf0, ra)

                @pl.when(ra + 2 < CHUNK)
                def _():
                    pltpu.async_copy(
                        emb_hbm.at[idx_v.at[ra + 2]], buf0, sem0)

                pltpu.make_async_copy(
                    emb_hbm.at[idx_v.at[rb]], buf1, sem1).wait()
                compute_row(buf1, rb)
                return 0

            lax.fori_loop(0, CHUNK // 2, pair_body, 0)
            pltpu.sync_copy(out_v, out_hbm.at[pl.ds(rbase, CHUNK)])
            return 0

        lax.fori_loop(0, ROWS_PER_W // CHUNK, chunk_body, 0)

    return k(x, emb, wf)


BLK = 2048


def _mlp(acc, W1, b1, W2, b2, wh, bf):
    def mk(acc_ref, w1_ref, b1_ref, w2_ref, b2_ref, wh_ref, bf_ref, o_ref):
        a = acc_ref[...]
        s = a[:, :D]
        q = a[:, D:2 * D]
        rv = a[:, 2 * D:]
        h = 0.5 * (s * s - q)
        h1 = jnp.maximum(
            jnp.dot(h, w1_ref[...], preferred_element_type=jnp.float32)
            + b1_ref[...], 0.0)
        h2 = jnp.maximum(
            jnp.dot(h1, w2_ref[...], preferred_element_type=jnp.float32)
            + b2_ref[...], 0.0)
        r = jnp.sum(rv, axis=1, keepdims=True)
        logit = (jnp.sum(h2 * wh_ref[...], axis=1, keepdims=True)
                 + r + bf_ref[...])
        o_ref[...] = 1.0 / (1.0 + jnp.exp(-logit))

    return pl.pallas_call(
        mk,
        grid=(B // BLK,),
        in_specs=[
            pl.BlockSpec((BLK, ACC_W), lambda i: (i, 0)),
            pl.BlockSpec((D, D // 2), lambda i: (0, 0)),
            pl.BlockSpec((1, D // 2), lambda i: (0, 0)),
            pl.BlockSpec((D // 2, D // 4), lambda i: (0, 0)),
            pl.BlockSpec((1, D // 4), lambda i: (0, 0)),
            pl.BlockSpec((1, D // 4), lambda i: (0, 0)),
            pl.BlockSpec((1, 1), lambda i: (0, 0)),
        ],
        out_specs=pl.BlockSpec((BLK, 1), lambda i: (i, 0)),
        out_shape=jax.ShapeDtypeStruct((B, 1), jnp.float32),
    )(acc, W1, b1.reshape(1, -1), W2, b2.reshape(1, -1), wh,
      bf.reshape(1, 1))


def kernel(x, Emb, W1, b1, W2, b2, Wf, bf):
    xi = x.astype(jnp.int32)
    wf_tok = Wf[D // 4:, 0].reshape(T, D)
    wh = Wf[:D // 4, 0].reshape(1, D // 4)
    acc = _sc_accumulate(xi, Emb, wf_tok)
    return _mlp(acc, W1, b1, W2, b2, wh, bf)
